# baseline (speedup 1.0000x reference)
import jax
import jax.numpy as jnp
from jax import lax
from jax.experimental import pallas as pl
from jax.experimental.pallas import tpu as pltpu
from jax.experimental.pallas import tpu_sc as plsc

N = 10000
E = 320000
D = 128
DE = 16

NC = 2
NS = 16
NW = NC * NS
EPW = E // NW
KC = 80
NCHUNK = EPW // KC
NPAD = 10240
RPT = NPAD // NS
ZROWS = 16
Z16R = 40
NV = D // 16
SQR = NS * 8
NB = 1000
GN = N // NB
EB = 4000
GE = E // EB


def _sc_body(u_hbm, dist_hbm, self_hbm, nbr_hbm,
             gu_out, sd_out, z_out, cnt_out, sq_out,
             gu_sh, sd_sh, z_sh, cnt_sh,
             selfi_a, nbri_a, dist_a, rows_a,
             selfi_b, nbri_b, dist_b,
             ones_v, zrow_v, z16_v, sq_v, sem_g, sem_l, sem_s):
    c = lax.axis_index("c")
    s = lax.axis_index("s")
    wid = c * NS + s
    base = s * RPT

    zeros16 = jnp.zeros((16,), jnp.float32)
    ones16 = jnp.ones((16,), jnp.float32)

    def init_ones(i, carry):
        ones_v[i, :] = ones16
        return carry
    lax.fori_loop(0, KC, init_ones, None)

    def init_zrow(i, carry):
        for j in range(NV):
            zrow_v[i, pl.ds(j * 16, 16)] = zeros16
        return carry
    lax.fori_loop(0, ZROWS, init_zrow, None)

    def init_z16(i, carry):
        z16_v[i, :] = zeros16
        return carry
    lax.fori_loop(0, Z16R, init_z16, None)

    def zero_gu(i, carry):
        pltpu.sync_copy(zrow_v, gu_sh.at[pl.ds(base + i * ZROWS, ZROWS)])
        return carry
    lax.fori_loop(0, RPT // ZROWS, zero_gu, None)

    def zero_16(i, carry):
        pltpu.sync_copy(z16_v, sd_sh.at[pl.ds(base + i * Z16R, Z16R)])
        pltpu.sync_copy(z16_v, z_sh.at[pl.ds(base + i * Z16R, Z16R)])
        pltpu.sync_copy(z16_v, cnt_sh.at[pl.ds(base + i * Z16R, Z16R)])
        return carry
    lax.fori_loop(0, RPT // Z16R, zero_16, None)

    plsc.subcore_barrier()

    bufs = ((selfi_a, nbri_a, dist_a),
            (selfi_b, nbri_b, dist_b))

    def fire_loads(t, b):
        si, ni, dv = bufs[b]
        pltpu.async_copy(self_hbm.at[wid, t], si, sem_l)
        pltpu.async_copy(nbr_hbm.at[wid, t], ni, sem_l)
        pltpu.async_copy(dist_hbm.at[wid, t], dv, sem_l)

    def drain_loads(b):
        si, ni, dv = bufs[b]
        pltpu.make_async_copy(self_hbm.at[wid, 0], si, sem_l).wait()
        pltpu.make_async_copy(nbr_hbm.at[wid, 0], ni, sem_l).wait()
        pltpu.make_async_copy(dist_hbm.at[wid, 0], dv, sem_l).wait()

    fire_loads(0, 0)

    def step(t, b, acc):
        si, ni, dv = bufs[b]
        drain_loads(b)
        pltpu.async_copy(u_hbm.at[ni], rows_a, sem_g).wait()

        @pl.when(t + 1 < NCHUNK)
        def _():
            fire_loads(t + 1, 1 - b)
        hs1 = pltpu.async_copy(rows_a, gu_sh.at[si], sem_s, add=True)
        hs2 = pltpu.async_copy(dv, sd_sh.at[si], sem_s, add=True)
        hs3 = pltpu.async_copy(dv, z_sh.at[ni], sem_s, add=True)
        hs4 = pltpu.async_copy(ones_v, cnt_sh.at[si], sem_s, add=True)

        def row(r, acc):
            return tuple(
                acc[k] + rows_a[r, pl.ds(k * 16, 16)] *
                rows_a[r, pl.ds(k * 16, 16)]
                for k in range(NV))
        acc = lax.fori_loop(0, KC, row, acc)
        hs1.wait()
        hs2.wait()
        hs3.wait()
        hs4.wait()
        return acc

    def outer(g, acc):
        acc = step(2 * g, 0, acc)
        return step(2 * g + 1, 1, acc)

    acc0 = tuple(jnp.zeros((16,), jnp.float32) for _ in range(NV))
    acc = lax.fori_loop(0, NCHUNK // 2, outer, acc0)
    if NCHUNK % 2:
        acc = step(NCHUNK - 1, 0, acc)
    for k in range(NV):
        sq_v[k, :] = acc[k]
    pltpu.sync_copy(sq_v, sq_out.at[c, pl.ds(s * 8, 8)])
    plsc.subcore_barrier()

    def out_gu(i, carry):
        pltpu.sync_copy(gu_sh.at[pl.ds(base + i * ZROWS, ZROWS)], zrow_v)
        pltpu.sync_copy(zrow_v,
                        gu_out.at[c, pl.ds(base + i * ZROWS, ZROWS)])
        return carry
    lax.fori_loop(0, RPT // ZROWS, out_gu, None)

    def out_16(i, carry):
        off = base + i * Z16R
        pltpu.sync_copy(sd_sh.at[pl.ds(off, Z16R)], z16_v)
        pltpu.sync_copy(z16_v, sd_out.at[c, pl.ds(off, Z16R)])
        pltpu.sync_copy(z_sh.at[pl.ds(off, Z16R)], z16_v)
        pltpu.sync_copy(z16_v, z_out.at[c, pl.ds(off, Z16R)])
        pltpu.sync_copy(cnt_sh.at[pl.ds(off, Z16R)], z16_v)
        pltpu.sync_copy(z16_v, cnt_out.at[c, pl.ds(off, Z16R)])
        return carry
    lax.fori_loop(0, RPT // Z16R, out_16, None)


_sc_segment = pl.kernel(
    _sc_body,
    out_type=[
        jax.ShapeDtypeStruct((NC, NPAD, D), jnp.float32),
        jax.ShapeDtypeStruct((NC, NPAD, DE), jnp.float32),
        jax.ShapeDtypeStruct((NC, NPAD, DE), jnp.float32),
        jax.ShapeDtypeStruct((NC, NPAD, DE), jnp.float32),
        jax.ShapeDtypeStruct((NC, SQR, DE), jnp.float32),
    ],
    mesh=plsc.VectorSubcoreMesh(core_axis_name="c", subcore_axis_name="s"),
    compiler_params=pltpu.CompilerParams(use_tc_tiling_on_sc=False),
    scratch_types=[
        pltpu.VMEM_SHARED((NPAD, D), jnp.float32),
        pltpu.VMEM_SHARED((NPAD, DE), jnp.float32),
        pltpu.VMEM_SHARED((NPAD, DE), jnp.float32),
        pltpu.VMEM_SHARED((NPAD, DE), jnp.float32),
        pltpu.VMEM((KC,), jnp.int32),
        pltpu.VMEM((KC,), jnp.int32),
        pltpu.VMEM((KC, DE), jnp.float32),
        pltpu.VMEM((KC, D), jnp.float32),
        pltpu.VMEM((KC,), jnp.int32),
        pltpu.VMEM((KC,), jnp.int32),
        pltpu.VMEM((KC, DE), jnp.float32),
        pltpu.VMEM((KC, DE), jnp.float32),
        pltpu.VMEM((ZROWS, D), jnp.float32),
        pltpu.VMEM((Z16R, DE), jnp.float32),
        pltpu.VMEM((8, DE), jnp.float32),
        pltpu.SemaphoreType.DMA,
        pltpu.SemaphoreType.DMA,
        pltpu.SemaphoreType.DMA,
    ],
)


def _u_body(atom_ref, w1a_ref, b1_ref, u_ref):
    u_ref[...] = lax.dot_general(
        atom_ref[...], w1a_ref[...], (((1,), (1,)), ((), ())),
        preferred_element_type=jnp.float32) + b1_ref[...]


def _vsq_body(d_ref, w1b_ref, o_ref):
    @pl.when(pl.program_id(0) == 0)
    def _():
        o_ref[...] = jnp.zeros_like(o_ref)
    v = lax.dot_general(d_ref[...], w1b_ref[...], (((1,), (1,)), ((), ())),
                        preferred_element_type=jnp.float32)
    o_ref[...] += jnp.sum(v * v, axis=0, keepdims=True)


def _epi_body(u_ref, atom_ref, gu_ref, z_ref, cnt_ref, sd_ref,
              susq_ref, vsq_ref, w1b_ref, w2_ref, b2_ref, g1_ref, be1_ref,
              g2_ref, be2_ref, o_ref,
              su_s, m2m_s, acol_s, s2_s, sdcol_s):
    p = pl.program_id(0)
    i = pl.program_id(1)

    @pl.when(jnp.logical_and(p == 0, i == 0))
    def _():
        su_s[...] = jnp.zeros_like(su_s)
        m2m_s[...] = jnp.zeros_like(m2m_s)
        acol_s[...] = jnp.zeros_like(acol_s)
        s2_s[...] = jnp.zeros_like(s2_s)
        sdcol_s[...] = jnp.zeros_like(sdcol_s)

    cnt = cnt_ref[0, :, 0:1] + cnt_ref[1, :, 0:1]
    mask = cnt > 0.0
    am = jnp.where(mask, atom_ref[...], 0.0)
    gu = gu_ref[0] + gu_ref[1]
    sd = sd_ref[0] + sd_ref[1]

    @pl.when(p == 0)
    def _():
        u = u_ref[...]
        su_s[...] += jnp.sum(gu, axis=0, keepdims=True)
        z = z_ref[0] + z_ref[1]
        m2m_s[...] += lax.dot_general(u, z, (((0,), (0,)), ((), ())),
                                      preferred_element_type=jnp.float32)
        acol_s[...] += jnp.sum(am, axis=0, keepdims=True)
        s2_s[...] += lax.dot_general(am, am, (((0,), (0,)), ((), ())),
                                     preferred_element_type=jnp.float32)
        sdcol_s[...] += jnp.sum(sd, axis=0, keepdims=True)
        o_ref[...] = jnp.zeros_like(o_ref)

    @pl.when(p == 1)
    def _():
        w1b = w1b_ref[...]
        w2 = w2_ref[...]
        b2 = b2_ref[...]
        inv_e = jnp.float32(1.0 / E)
        m1 = (su_s[...] + lax.dot_general(
            sdcol_s[...], w1b, (((1,), (1,)), ((), ())),
            preferred_element_type=jnp.float32)) * inv_e
        cross = jnp.sum(w1b * m2m_s[...], axis=1)[None, :]
        et2 = (susq_ref[...] + 2.0 * cross + vsq_ref[...]) * inv_e
        v1 = et2 - m1 * m1
        s1 = g1_ref[...] * lax.rsqrt(v1 + 1e-5)
        t1 = be1_ref[...] - m1 * s1

        mu_a = acol_s[...] * jnp.float32(1.0 / N)
        pv = lax.dot_general(mu_a, w2, (((1,), (1,)), ((), ())),
                             preferred_element_type=jnp.float32)
        t_mat = lax.dot_general(w2, s2_s[...] * jnp.float32(1.0 / N),
                                (((1,), (0,)), ((), ())),
                                preferred_element_type=jnp.float32)
        q = jnp.sum(t_mat * w2, axis=1)[None, :]
        v2 = q - pv * pv
        s2c = g2_ref[...] * lax.rsqrt(v2 + 1e-5)
        t2c = be2_ref[...] - (pv + b2) * s2c

        sdw = lax.dot_general(sd, w1b, (((1,), (1,)), ((), ())),
                              preferred_element_type=jnp.float32)
        fea = jnp.where(mask, (gu + sdw) / jnp.maximum(cnt, 1.0) * s1 + t1,
                        0.0)
        y = lax.dot_general(am, w2, (((1,), (1,)), ((), ())),
                            preferred_element_type=jnp.float32) + b2
        x = y * s2c + t2c + fea
        o_ref[...] = jnp.maximum(x, 0.0) + jnp.log1p(jnp.exp(-jnp.abs(x)))


_full = pl.BlockSpec((1, D), lambda i: (0, 0))
_full16 = pl.BlockSpec((1, DE), lambda i: (0, 0))
_blkN = pl.BlockSpec((NB, D), lambda i: (i, 0))
_blk3 = pl.BlockSpec((NC, NB, D), lambda i: (0, i, 0))
_blk3_16 = pl.BlockSpec((NC, NB, DE), lambda i: (0, i, 0))


_u_call = pl.pallas_call(
    _u_body,
    grid=(GN,),
    in_specs=[_blkN,
              pl.BlockSpec((D, D), lambda i: (0, 0)),
              _full],
    out_specs=_blkN,
    out_shape=jax.ShapeDtypeStruct((N, D), jnp.float32),
)

_vsq_call = pl.pallas_call(
    _vsq_body,
    grid=(GE,),
    in_specs=[pl.BlockSpec((EB, DE), lambda i: (i, 0)),
              pl.BlockSpec((D, DE), lambda i: (0, 0))],
    out_specs=_full,
    out_shape=jax.ShapeDtypeStruct((1, D), jnp.float32),
)

_epi_call = pl.pallas_call(
    _epi_body,
    grid=(2, GN),
    in_specs=[pl.BlockSpec((NB, D), lambda p, i: (i, 0)),
              pl.BlockSpec((NB, D), lambda p, i: (i, 0)),
              pl.BlockSpec((NC, NB, D), lambda p, i: (0, i, 0)),
              pl.BlockSpec((NC, NB, DE), lambda p, i: (0, i, 0)),
              pl.BlockSpec((NC, NB, DE), lambda p, i: (0, i, 0)),
              pl.BlockSpec((NC, NB, DE), lambda p, i: (0, i, 0)),
              pl.BlockSpec((1, D), lambda p, i: (0, 0)),
              pl.BlockSpec((1, D), lambda p, i: (0, 0)),
              pl.BlockSpec((D, DE), lambda p, i: (0, 0)),
              pl.BlockSpec((D, D), lambda p, i: (0, 0)),
              pl.BlockSpec((1, D), lambda p, i: (0, 0)),
              pl.BlockSpec((1, D), lambda p, i: (0, 0)),
              pl.BlockSpec((1, D), lambda p, i: (0, 0)),
              pl.BlockSpec((1, D), lambda p, i: (0, 0)),
              pl.BlockSpec((1, D), lambda p, i: (0, 0))],
    out_specs=pl.BlockSpec((NB, D), lambda p, i: (i, 0)),
    out_shape=jax.ShapeDtypeStruct((N, D), jnp.float32),
    scratch_shapes=[pltpu.VMEM((1, D), jnp.float32),
                    pltpu.VMEM((D, DE), jnp.float32),
                    pltpu.VMEM((1, D), jnp.float32),
                    pltpu.VMEM((D, D), jnp.float32),
                    pltpu.VMEM((1, DE), jnp.float32)],
)


@jax.jit
def _run(atom_fea, nbr_dist_fea, self_fea_idx, nbr_fea_idx,
         W1, b1, g1, be1, W2, b2, g2, be2):
    w1a = W1[:, :D]
    w1b = W1[:, D:]
    u = _u_call(atom_fea, w1a, b1.reshape(1, D))
    vsq = _vsq_call(nbr_dist_fea, w1b)
    self_r = self_fea_idx.reshape(NW, NCHUNK, KC)
    nbr_r = nbr_fea_idx.reshape(NW, NCHUNK, KC)
    dist_r = nbr_dist_fea.reshape(NW, NCHUNK, KC, DE)
    gu2, sd2, z2, cnt2, sq2 = _sc_segment(u, dist_r, self_r, nbr_r)
    susq = sq2.reshape(NC, NS, 8, DE).sum(axis=(0, 1)).reshape(1, D)
    return _epi_call(u, atom_fea, gu2, z2, cnt2, sd2, susq, vsq, w1b, W2,
                     b2.reshape(1, D), g1.reshape(1, D), be1.reshape(1, D),
                     g2.reshape(1, D), be2.reshape(1, D))


def kernel(atom_fea, nbr_dist_fea, nbr_adj_value, nbr_bond_type,
           self_fea_idx, nbr_fea_idx, ads_atom_idx,
           W1, b1, g1, be1, W2, b2, g2, be2):
    del nbr_adj_value, nbr_bond_type, ads_atom_idx
    return _run(atom_fea, nbr_dist_fea, self_fea_idx, nbr_fea_idx,
                W1, b1, g1, be1, W2, b2, g2, be2)


# speedup vs baseline: 1.0062x; 1.0062x over previous
import jax
import jax.numpy as jnp
from jax import lax
from jax.experimental import pallas as pl
from jax.experimental.pallas import tpu as pltpu
from jax.experimental.pallas import tpu_sc as plsc

N = 10000
E = 320000
D = 128
DE = 16

NC = 2
NS = 16
NW = NC * NS
EPW = E // NW
KC = 80
NCHUNK = EPW // KC
NPAD = 10240
RPT = NPAD // NS
ZROWS = 16
Z16R = 40
NV = D // 16
SQR = NS * 8
NB = 1000
GN = N // NB
EB = 4000
GE = E // EB


def _sc_body(u_hbm, dist_hbm, self_hbm, nbr_hbm,
             gu_out, sd_out, z_out, cnt_out, sq_out,
             gu_sh, sd_sh, z_sh, cnt_sh,
             selfi_a, nbri_a, dist_a, rows_a,
             selfi_b, nbri_b, dist_b,
             ones_v, zrow_v, z16_v, sq_v, sem_g, sem_l, sem_s):
    c = lax.axis_index("c")
    s = lax.axis_index("s")
    wid = c * NS + s
    base = s * RPT

    zeros16 = jnp.zeros((16,), jnp.float32)
    ones16 = jnp.ones((16,), jnp.float32)

    def init_ones(i, carry):
        ones_v[i, :] = ones16
        return carry
    lax.fori_loop(0, KC, init_ones, None)

    def init_zrow(i, carry):
        for j in range(NV):
            zrow_v[i, pl.ds(j * 16, 16)] = zeros16
        return carry
    lax.fori_loop(0, ZROWS, init_zrow, None)

    def init_z16(i, carry):
        z16_v[i, :] = zeros16
        return carry
    lax.fori_loop(0, Z16R, init_z16, None)

    def zero_gu(i, carry):
        pltpu.sync_copy(zrow_v, gu_sh.at[pl.ds(base + i * ZROWS, ZROWS)])
        return carry
    lax.fori_loop(0, RPT // ZROWS, zero_gu, None)

    def zero_16(i, carry):
        pltpu.sync_copy(z16_v, sd_sh.at[pl.ds(base + i * Z16R, Z16R)])
        pltpu.sync_copy(z16_v, z_sh.at[pl.ds(base + i * Z16R, Z16R)])
        pltpu.sync_copy(z16_v, cnt_sh.at[pl.ds(base + i * Z16R, Z16R)])
        return carry
    lax.fori_loop(0, RPT // Z16R, zero_16, None)

    plsc.subcore_barrier()

    bufs = ((selfi_a, nbri_a, dist_a),
            (selfi_b, nbri_b, dist_b))

    def fire_loads(t, b):
        si, ni, dv = bufs[b]
        pltpu.async_copy(self_hbm.at[wid, t], si, sem_l)
        pltpu.async_copy(nbr_hbm.at[wid, t], ni, sem_l)
        pltpu.async_copy(dist_hbm.at[wid, t], dv, sem_l)

    def drain_loads(b):
        si, ni, dv = bufs[b]
        pltpu.make_async_copy(self_hbm.at[wid, 0], si, sem_l).wait()
        pltpu.make_async_copy(nbr_hbm.at[wid, 0], ni, sem_l).wait()
        pltpu.make_async_copy(dist_hbm.at[wid, 0], dv, sem_l).wait()

    fire_loads(0, 0)

    def step(t, b, acc):
        si, ni, dv = bufs[b]
        drain_loads(b)
        pltpu.async_copy(u_hbm.at[ni], rows_a, sem_g).wait()

        @pl.when(t + 1 < NCHUNK)
        def _():
            fire_loads(t + 1, 1 - b)
        hs1 = pltpu.async_copy(rows_a, gu_sh.at[si], sem_s, add=True)
        hs2 = pltpu.async_copy(dv, sd_sh.at[si], sem_s, add=True)
        hs3 = pltpu.async_copy(dv, z_sh.at[ni], sem_s, add=True)
        hs4 = pltpu.async_copy(ones_v, cnt_sh.at[si], sem_s, add=True)

        def row(r, acc):
            return tuple(
                acc[k] + rows_a[r, pl.ds(k * 16, 16)] *
                rows_a[r, pl.ds(k * 16, 16)]
                for k in range(NV))
        acc = lax.fori_loop(0, KC, row, acc)
        hs1.wait()
        hs2.wait()
        hs3.wait()
        hs4.wait()
        return acc

    def outer(g, acc):
        acc = step(2 * g, 0, acc)
        return step(2 * g + 1, 1, acc)

    acc0 = tuple(jnp.zeros((16,), jnp.float32) for _ in range(NV))
    acc = lax.fori_loop(0, NCHUNK // 2, outer, acc0)
    if NCHUNK % 2:
        acc = step(NCHUNK - 1, 0, acc)
    for k in range(NV):
        sq_v[k, :] = acc[k]
    pltpu.sync_copy(sq_v, sq_out.at[c, pl.ds(s * 8, 8)])
    plsc.subcore_barrier()

    def out_gu(i, carry):
        pltpu.sync_copy(gu_sh.at[pl.ds(base + i * ZROWS, ZROWS)], zrow_v)
        pltpu.sync_copy(zrow_v,
                        gu_out.at[c, pl.ds(base + i * ZROWS, ZROWS)])
        return carry
    lax.fori_loop(0, RPT // ZROWS, out_gu, None)

    def out_16(i, carry):
        off = base + i * Z16R
        pltpu.sync_copy(sd_sh.at[pl.ds(off, Z16R)], z16_v)
        pltpu.sync_copy(z16_v, sd_out.at[c, pl.ds(off, Z16R)])
        pltpu.sync_copy(z_sh.at[pl.ds(off, Z16R)], z16_v)
        pltpu.sync_copy(z16_v, z_out.at[c, pl.ds(off, Z16R)])
        pltpu.sync_copy(cnt_sh.at[pl.ds(off, Z16R)], z16_v)
        pltpu.sync_copy(z16_v, cnt_out.at[c, pl.ds(off, Z16R)])
        return carry
    lax.fori_loop(0, RPT // Z16R, out_16, None)


_sc_segment = pl.kernel(
    _sc_body,
    out_type=[
        jax.ShapeDtypeStruct((NC, NPAD, D), jnp.float32),
        jax.ShapeDtypeStruct((NC, NPAD, DE), jnp.float32),
        jax.ShapeDtypeStruct((NC, NPAD, DE), jnp.float32),
        jax.ShapeDtypeStruct((NC, NPAD, DE), jnp.float32),
        jax.ShapeDtypeStruct((NC, SQR, DE), jnp.float32),
    ],
    mesh=plsc.VectorSubcoreMesh(core_axis_name="c", subcore_axis_name="s"),
    compiler_params=pltpu.CompilerParams(use_tc_tiling_on_sc=False),
    scratch_types=[
        pltpu.VMEM_SHARED((NPAD, D), jnp.float32),
        pltpu.VMEM_SHARED((NPAD, DE), jnp.float32),
        pltpu.VMEM_SHARED((NPAD, DE), jnp.float32),
        pltpu.VMEM_SHARED((NPAD, DE), jnp.float32),
        pltpu.VMEM((KC,), jnp.int32),
        pltpu.VMEM((KC,), jnp.int32),
        pltpu.VMEM((KC, DE), jnp.float32),
        pltpu.VMEM((KC, D), jnp.float32),
        pltpu.VMEM((KC,), jnp.int32),
        pltpu.VMEM((KC,), jnp.int32),
        pltpu.VMEM((KC, DE), jnp.float32),
        pltpu.VMEM((KC, DE), jnp.float32),
        pltpu.VMEM((ZROWS, D), jnp.float32),
        pltpu.VMEM((Z16R, DE), jnp.float32),
        pltpu.VMEM((8, DE), jnp.float32),
        pltpu.SemaphoreType.DMA,
        pltpu.SemaphoreType.DMA,
        pltpu.SemaphoreType.DMA,
    ],
)


def _u_body(atom_ref, w1a_ref, b1_ref, u_ref):
    u_ref[...] = lax.dot_general(
        atom_ref[...], w1a_ref[...], (((1,), (1,)), ((), ())),
        preferred_element_type=jnp.float32) + b1_ref[...]


def _vsq_body(d_ref, w1b_ref, o_ref):
    @pl.when(pl.program_id(0) == 0)
    def _():
        o_ref[...] = jnp.zeros_like(o_ref)
    v = lax.dot_general(d_ref[...], w1b_ref[...], (((1,), (1,)), ((), ())),
                        preferred_element_type=jnp.float32)
    o_ref[...] += jnp.sum(v * v, axis=0, keepdims=True)


def _stats_body(u_ref, atom_ref, gu_ref, z_ref, cnt_ref, sd_ref,
                su_ref, m2m_ref, acol_ref, s2_ref, sdcol_ref):
    @pl.when(pl.program_id(0) == 0)
    def _():
        su_ref[...] = jnp.zeros_like(su_ref)
        m2m_ref[...] = jnp.zeros_like(m2m_ref)
        acol_ref[...] = jnp.zeros_like(acol_ref)
        s2_ref[...] = jnp.zeros_like(s2_ref)
        sdcol_ref[...] = jnp.zeros_like(sdcol_ref)
    u = u_ref[...]
    gu = gu_ref[0] + gu_ref[1]
    su_ref[...] += jnp.sum(gu, axis=0, keepdims=True)
    z = z_ref[0] + z_ref[1]
    m2m_ref[...] += lax.dot_general(u, z, (((0,), (0,)), ((), ())),
                                    preferred_element_type=jnp.float32)
    cnt = cnt_ref[0, :, 0:1] + cnt_ref[1, :, 0:1]
    am = jnp.where(cnt > 0.0, atom_ref[...], 0.0)
    acol_ref[...] += jnp.sum(am, axis=0, keepdims=True)
    s2_ref[...] += lax.dot_general(am, am, (((0,), (0,)), ((), ())),
                                   preferred_element_type=jnp.float32)
    sdcol_ref[...] += jnp.sum(sd_ref[0] + sd_ref[1], axis=0, keepdims=True)


def _final_body(gu_ref, sd_ref, cnt_ref, atom_ref,
                su_ref, susq_ref, m2m_ref, acol_ref, s2_ref, sdcol_ref,
                vsq_ref, w1b_ref, w2_ref, b2_ref, g1_ref, be1_ref,
                g2_ref, be2_ref, o_ref):
    w1b = w1b_ref[...]
    w2 = w2_ref[...]
    b2 = b2_ref[...]
    inv_e = jnp.float32(1.0 / E)
    m1 = (su_ref[...] + lax.dot_general(
        sdcol_ref[...], w1b, (((1,), (1,)), ((), ())),
        preferred_element_type=jnp.float32)) * inv_e
    susq = susq_ref[...]
    cross = jnp.sum(w1b * m2m_ref[...], axis=1)[None, :]
    et2 = (susq + 2.0 * cross + vsq_ref[...]) * inv_e
    v1 = et2 - m1 * m1
    s1 = g1_ref[...] * lax.rsqrt(v1 + 1e-5)
    t1 = be1_ref[...] - m1 * s1

    mu_a = acol_ref[...] * jnp.float32(1.0 / N)
    p = lax.dot_general(mu_a, w2, (((1,), (1,)), ((), ())),
                        preferred_element_type=jnp.float32)
    t_mat = lax.dot_general(w2, s2_ref[...] * jnp.float32(1.0 / N),
                            (((1,), (0,)), ((), ())),
                            preferred_element_type=jnp.float32)
    q = jnp.sum(t_mat * w2, axis=1)[None, :]
    v2 = q - p * p
    s2c = g2_ref[...] * lax.rsqrt(v2 + 1e-5)
    t2c = be2_ref[...] - (p + b2) * s2c

    cnt = cnt_ref[0, :, 0:1] + cnt_ref[1, :, 0:1]
    mask = cnt > 0.0
    gu = gu_ref[0] + gu_ref[1]
    sdw = lax.dot_general(sd_ref[0] + sd_ref[1], w1b, (((1,), (1,)), ((), ())),
                          preferred_element_type=jnp.float32)
    fea = jnp.where(mask, (gu + sdw) / jnp.maximum(cnt, 1.0) * s1 + t1, 0.0)
    am = jnp.where(mask, atom_ref[...], 0.0)
    y = lax.dot_general(am, w2, (((1,), (1,)), ((), ())),
                        preferred_element_type=jnp.float32) + b2
    x = y * s2c + t2c + fea
    o_ref[...] = jnp.maximum(x, 0.0) + jnp.log1p(jnp.exp(-jnp.abs(x)))


_full = pl.BlockSpec((1, D), lambda i: (0, 0))
_full16 = pl.BlockSpec((1, DE), lambda i: (0, 0))
_blkN = pl.BlockSpec((NB, D), lambda i: (i, 0))
_blk3 = pl.BlockSpec((NC, NB, D), lambda i: (0, i, 0))
_blk3_16 = pl.BlockSpec((NC, NB, DE), lambda i: (0, i, 0))


_u_call = pl.pallas_call(
    _u_body,
    grid=(GN,),
    in_specs=[_blkN,
              pl.BlockSpec((D, D), lambda i: (0, 0)),
              _full],
    out_specs=_blkN,
    out_shape=jax.ShapeDtypeStruct((N, D), jnp.float32),
)

_vsq_call = pl.pallas_call(
    _vsq_body,
    grid=(GE,),
    in_specs=[pl.BlockSpec((EB, DE), lambda i: (i, 0)),
              pl.BlockSpec((D, DE), lambda i: (0, 0))],
    out_specs=_full,
    out_shape=jax.ShapeDtypeStruct((1, D), jnp.float32),
)

_stats_call = pl.pallas_call(
    _stats_body,
    grid=(GN,),
    in_specs=[_blkN, _blkN, _blk3, _blk3_16, _blk3_16, _blk3_16],
    out_specs=[_full,
               pl.BlockSpec((D, DE), lambda i: (0, 0)),
               _full,
               pl.BlockSpec((D, D), lambda i: (0, 0)),
               _full16],
    out_shape=[jax.ShapeDtypeStruct((1, D), jnp.float32),
               jax.ShapeDtypeStruct((D, DE), jnp.float32),
               jax.ShapeDtypeStruct((1, D), jnp.float32),
               jax.ShapeDtypeStruct((D, D), jnp.float32),
               jax.ShapeDtypeStruct((1, DE), jnp.float32)],
)

_final_call = pl.pallas_call(
    _final_body,
    grid=(GN,),
    in_specs=[_blk3, _blk3_16, _blk3_16, _blkN,
              _full, _full,
              pl.BlockSpec((D, DE), lambda i: (0, 0)),
              _full,
              pl.BlockSpec((D, D), lambda i: (0, 0)),
              _full16, _full,
              pl.BlockSpec((D, DE), lambda i: (0, 0)),
              pl.BlockSpec((D, D), lambda i: (0, 0)),
              _full, _full, _full, _full, _full],
    out_specs=_blkN,
    out_shape=jax.ShapeDtypeStruct((N, D), jnp.float32),
)


@jax.jit
def _run(atom_fea, nbr_dist_fea, self_fea_idx, nbr_fea_idx,
         W1, b1, g1, be1, W2, b2, g2, be2):
    w1a = W1[:, :D]
    w1b = W1[:, D:]
    u = _u_call(atom_fea, w1a, b1.reshape(1, D))
    vsq = _vsq_call(nbr_dist_fea, w1b)
    self_r = self_fea_idx.reshape(NW, NCHUNK, KC)
    nbr_r = nbr_fea_idx.reshape(NW, NCHUNK, KC)
    dist_r = nbr_dist_fea.reshape(NW, NCHUNK, KC, DE)
    gu2, sd2, z2, cnt2, sq2 = _sc_segment(u, dist_r, self_r, nbr_r)
    susq = sq2.reshape(NC, NS, 8, DE).sum(axis=(0, 1)).reshape(1, D)
    su, m2m, acol, s2g, sdcol = _stats_call(u, atom_fea, gu2, z2, cnt2, sd2)
    return _final_call(gu2, sd2, cnt2, atom_fea, su, susq, m2m, acol, s2g,
                       sdcol, vsq, w1b, W2, b2.reshape(1, D),
                       g1.reshape(1, D), be1.reshape(1, D),
                       g2.reshape(1, D), be2.reshape(1, D))


def kernel(atom_fea, nbr_dist_fea, nbr_adj_value, nbr_bond_type,
           self_fea_idx, nbr_fea_idx, ads_atom_idx,
           W1, b1, g1, be1, W2, b2, g2, be2):
    del nbr_adj_value, nbr_bond_type, ads_atom_idx
    return _run(atom_fea, nbr_dist_fea, self_fea_idx, nbr_fea_idx,
                W1, b1, g1, be1, W2, b2, g2, be2)


# speedup vs baseline: 1.0202x; 1.0140x over previous
import jax
import jax.numpy as jnp
from jax import lax
from jax.experimental import pallas as pl
from jax.experimental.pallas import tpu as pltpu
from jax.experimental.pallas import tpu_sc as plsc

N = 10000
E = 320000
D = 128
DE = 16

NC = 2
NS = 16
NW = NC * NS
EPW = E // NW
KC = 80
NCHUNK = EPW // KC
NPAD = 10240
RPT = NPAD // NS
ZROWS = 16
Z16R = 40
NV = D // 16
SQR = NS * 8
NB = 1000
GN = N // NB
EB = 4000
GE = E // EB


def _sc_body(u_hbm, dist_hbm, self_hbm, nbr_hbm,
             gu_out, sd_out, z_out, cnt_out, sq_out,
             gu_sh, sd_sh, z_sh, cnt_sh,
             selfi_a, nbri_a, dist_a, rows_a,
             selfi_b, nbri_b, dist_b,
             ones_v, zrow_v, z16_v, sq_v, sem_g, sem_l, sem_s):
    c = lax.axis_index("c")
    s = lax.axis_index("s")
    wid = c * NS + s
    base = s * RPT

    zeros16 = jnp.zeros((16,), jnp.float32)
    ones16 = jnp.ones((16,), jnp.float32)

    def init_ones(i, carry):
        ones_v[i, :] = ones16
        return carry
    lax.fori_loop(0, KC, init_ones, None)

    def init_zrow(i, carry):
        for j in range(NV):
            zrow_v[i, pl.ds(j * 16, 16)] = zeros16
        return carry
    lax.fori_loop(0, ZROWS, init_zrow, None)

    def init_z16(i, carry):
        z16_v[i, :] = zeros16
        return carry
    lax.fori_loop(0, Z16R, init_z16, None)

    def zero_gu(i, carry):
        pltpu.sync_copy(zrow_v, gu_sh.at[pl.ds(base + i * ZROWS, ZROWS)])
        return carry
    lax.fori_loop(0, RPT // ZROWS, zero_gu, None)

    def zero_16(i, carry):
        pltpu.sync_copy(z16_v, sd_sh.at[pl.ds(base + i * Z16R, Z16R)])
        pltpu.sync_copy(z16_v, z_sh.at[pl.ds(base + i * Z16R, Z16R)])
        pltpu.sync_copy(z16_v, cnt_sh.at[pl.ds(base + i * Z16R, Z16R)])
        return carry
    lax.fori_loop(0, RPT // Z16R, zero_16, None)

    plsc.subcore_barrier()

    bufs = ((selfi_a, nbri_a, dist_a),
            (selfi_b, nbri_b, dist_b))

    def fire_loads(t, b):
        si, ni, dv = bufs[b]
        pltpu.async_copy(self_hbm.at[wid, t], si, sem_l)
        pltpu.async_copy(nbr_hbm.at[wid, t], ni, sem_l)
        pltpu.async_copy(dist_hbm.at[wid, t], dv, sem_l)

    def drain_loads(b):
        si, ni, dv = bufs[b]
        pltpu.make_async_copy(self_hbm.at[wid, 0], si, sem_l).wait()
        pltpu.make_async_copy(nbr_hbm.at[wid, 0], ni, sem_l).wait()
        pltpu.make_async_copy(dist_hbm.at[wid, 0], dv, sem_l).wait()

    fire_loads(0, 0)
    drain_loads(0)
    pltpu.async_copy(u_hbm.at[nbri_a], rows_a, sem_g)

    def step(t, b, acc):
        si, ni, dv = bufs[b]
        pltpu.make_async_copy(u_hbm.at[ni], rows_a, sem_g).wait()
        hs1 = pltpu.async_copy(rows_a, gu_sh.at[si], sem_s, add=True)
        hs2 = pltpu.async_copy(dv, sd_sh.at[si], sem_s, add=True)
        hs3 = pltpu.async_copy(dv, z_sh.at[ni], sem_s, add=True)
        hs4 = pltpu.async_copy(ones_v, cnt_sh.at[si], sem_s, add=True)

        @pl.when(t + 1 < NCHUNK)
        def _():
            fire_loads(t + 1, 1 - b)

        def row(r, acc):
            return tuple(
                acc[k] + rows_a[r, pl.ds(k * 16, 16)] *
                rows_a[r, pl.ds(k * 16, 16)]
                for k in range(NV))
        acc = lax.fori_loop(0, KC, row, acc)
        hs1.wait()

        @pl.when(t + 1 < NCHUNK)
        def _():
            drain_loads(1 - b)
            nsi, nni, ndv = bufs[1 - b]
            pltpu.async_copy(u_hbm.at[nni], rows_a, sem_g)
        hs2.wait()
        hs3.wait()
        hs4.wait()
        return acc

    def outer(g, acc):
        acc = step(2 * g, 0, acc)
        return step(2 * g + 1, 1, acc)

    acc0 = tuple(jnp.zeros((16,), jnp.float32) for _ in range(NV))
    acc = lax.fori_loop(0, NCHUNK // 2, outer, acc0)
    if NCHUNK % 2:
        acc = step(NCHUNK - 1, 0, acc)
    for k in range(NV):
        sq_v[k, :] = acc[k]
    pltpu.sync_copy(sq_v, sq_out.at[c, pl.ds(s * 8, 8)])
    plsc.subcore_barrier()

    def out_gu(i, carry):
        pltpu.sync_copy(gu_sh.at[pl.ds(base + i * ZROWS, ZROWS)], zrow_v)
        pltpu.sync_copy(zrow_v,
                        gu_out.at[c, pl.ds(base + i * ZROWS, ZROWS)])
        return carry
    lax.fori_loop(0, RPT // ZROWS, out_gu, None)

    def out_16(i, carry):
        off = base + i * Z16R
        pltpu.sync_copy(sd_sh.at[pl.ds(off, Z16R)], z16_v)
        pltpu.sync_copy(z16_v, sd_out.at[c, pl.ds(off, Z16R)])
        pltpu.sync_copy(z_sh.at[pl.ds(off, Z16R)], z16_v)
        pltpu.sync_copy(z16_v, z_out.at[c, pl.ds(off, Z16R)])
        pltpu.sync_copy(cnt_sh.at[pl.ds(off, Z16R)], z16_v)
        pltpu.sync_copy(z16_v, cnt_out.at[c, pl.ds(off, Z16R)])
        return carry
    lax.fori_loop(0, RPT // Z16R, out_16, None)


_sc_segment = pl.kernel(
    _sc_body,
    out_type=[
        jax.ShapeDtypeStruct((NC, NPAD, D), jnp.float32),
        jax.ShapeDtypeStruct((NC, NPAD, DE), jnp.float32),
        jax.ShapeDtypeStruct((NC, NPAD, DE), jnp.float32),
        jax.ShapeDtypeStruct((NC, NPAD, DE), jnp.float32),
        jax.ShapeDtypeStruct((NC, SQR, DE), jnp.float32),
    ],
    mesh=plsc.VectorSubcoreMesh(core_axis_name="c", subcore_axis_name="s"),
    compiler_params=pltpu.CompilerParams(use_tc_tiling_on_sc=False),
    scratch_types=[
        pltpu.VMEM_SHARED((NPAD, D), jnp.float32),
        pltpu.VMEM_SHARED((NPAD, DE), jnp.float32),
        pltpu.VMEM_SHARED((NPAD, DE), jnp.float32),
        pltpu.VMEM_SHARED((NPAD, DE), jnp.float32),
        pltpu.VMEM((KC,), jnp.int32),
        pltpu.VMEM((KC,), jnp.int32),
        pltpu.VMEM((KC, DE), jnp.float32),
        pltpu.VMEM((KC, D), jnp.float32),
        pltpu.VMEM((KC,), jnp.int32),
        pltpu.VMEM((KC,), jnp.int32),
        pltpu.VMEM((KC, DE), jnp.float32),
        pltpu.VMEM((KC, DE), jnp.float32),
        pltpu.VMEM((ZROWS, D), jnp.float32),
        pltpu.VMEM((Z16R, DE), jnp.float32),
        pltpu.VMEM((8, DE), jnp.float32),
        pltpu.SemaphoreType.DMA,
        pltpu.SemaphoreType.DMA,
        pltpu.SemaphoreType.DMA,
    ],
)


def _u_body(atom_ref, w1a_ref, b1_ref, u_ref):
    u_ref[...] = lax.dot_general(
        atom_ref[...], w1a_ref[...], (((1,), (1,)), ((), ())),
        preferred_element_type=jnp.float32) + b1_ref[...]


def _vsq_body(d_ref, w1b_ref, o_ref):
    @pl.when(pl.program_id(0) == 0)
    def _():
        o_ref[...] = jnp.zeros_like(o_ref)
    v = lax.dot_general(d_ref[...], w1b_ref[...], (((1,), (1,)), ((), ())),
                        preferred_element_type=jnp.float32)
    o_ref[...] += jnp.sum(v * v, axis=0, keepdims=True)


def _stats_body(u_ref, atom_ref, gu_ref, z_ref, cnt_ref, sd_ref,
                su_ref, m2m_ref, acol_ref, s2_ref, sdcol_ref):
    @pl.when(pl.program_id(0) == 0)
    def _():
        su_ref[...] = jnp.zeros_like(su_ref)
        m2m_ref[...] = jnp.zeros_like(m2m_ref)
        acol_ref[...] = jnp.zeros_like(acol_ref)
        s2_ref[...] = jnp.zeros_like(s2_ref)
        sdcol_ref[...] = jnp.zeros_like(sdcol_ref)
    u = u_ref[...]
    gu = gu_ref[0] + gu_ref[1]
    su_ref[...] += jnp.sum(gu, axis=0, keepdims=True)
    z = z_ref[0] + z_ref[1]
    m2m_ref[...] += lax.dot_general(u, z, (((0,), (0,)), ((), ())),
                                    preferred_element_type=jnp.float32)
    cnt = cnt_ref[0, :, 0:1] + cnt_ref[1, :, 0:1]
    am = jnp.where(cnt > 0.0, atom_ref[...], 0.0)
    acol_ref[...] += jnp.sum(am, axis=0, keepdims=True)
    s2_ref[...] += lax.dot_general(am, am, (((0,), (0,)), ((), ())),
                                   preferred_element_type=jnp.float32)
    sdcol_ref[...] += jnp.sum(sd_ref[0] + sd_ref[1], axis=0, keepdims=True)


def _final_body(gu_ref, sd_ref, cnt_ref, atom_ref,
                su_ref, susq_ref, m2m_ref, acol_ref, s2_ref, sdcol_ref,
                vsq_ref, w1b_ref, w2_ref, b2_ref, g1_ref, be1_ref,
                g2_ref, be2_ref, o_ref):
    w1b = w1b_ref[...]
    w2 = w2_ref[...]
    b2 = b2_ref[...]
    inv_e = jnp.float32(1.0 / E)
    m1 = (su_ref[...] + lax.dot_general(
        sdcol_ref[...], w1b, (((1,), (1,)), ((), ())),
        preferred_element_type=jnp.float32)) * inv_e
    susq = susq_ref[...]
    cross = jnp.sum(w1b * m2m_ref[...], axis=1)[None, :]
    et2 = (susq + 2.0 * cross + vsq_ref[...]) * inv_e
    v1 = et2 - m1 * m1
    s1 = g1_ref[...] * lax.rsqrt(v1 + 1e-5)
    t1 = be1_ref[...] - m1 * s1

    mu_a = acol_ref[...] * jnp.float32(1.0 / N)
    p = lax.dot_general(mu_a, w2, (((1,), (1,)), ((), ())),
                        preferred_element_type=jnp.float32)
    t_mat = lax.dot_general(w2, s2_ref[...] * jnp.float32(1.0 / N),
                            (((1,), (0,)), ((), ())),
                            preferred_element_type=jnp.float32)
    q = jnp.sum(t_mat * w2, axis=1)[None, :]
    v2 = q - p * p
    s2c = g2_ref[...] * lax.rsqrt(v2 + 1e-5)
    t2c = be2_ref[...] - (p + b2) * s2c

    cnt = cnt_ref[0, :, 0:1] + cnt_ref[1, :, 0:1]
    mask = cnt > 0.0
    gu = gu_ref[0] + gu_ref[1]
    sdw = lax.dot_general(sd_ref[0] + sd_ref[1], w1b, (((1,), (1,)), ((), ())),
                          preferred_element_type=jnp.float32)
    fea = jnp.where(mask, (gu + sdw) / jnp.maximum(cnt, 1.0) * s1 + t1, 0.0)
    am = jnp.where(mask, atom_ref[...], 0.0)
    y = lax.dot_general(am, w2, (((1,), (1,)), ((), ())),
                        preferred_element_type=jnp.float32) + b2
    x = y * s2c + t2c + fea
    o_ref[...] = jnp.maximum(x, 0.0) + jnp.log1p(jnp.exp(-jnp.abs(x)))


_full = pl.BlockSpec((1, D), lambda i: (0, 0))
_full16 = pl.BlockSpec((1, DE), lambda i: (0, 0))
_blkN = pl.BlockSpec((NB, D), lambda i: (i, 0))
_blk3 = pl.BlockSpec((NC, NB, D), lambda i: (0, i, 0))
_blk3_16 = pl.BlockSpec((NC, NB, DE), lambda i: (0, i, 0))


_u_call = pl.pallas_call(
    _u_body,
    grid=(GN,),
    in_specs=[_blkN,
              pl.BlockSpec((D, D), lambda i: (0, 0)),
              _full],
    out_specs=_blkN,
    out_shape=jax.ShapeDtypeStruct((N, D), jnp.float32),
)

_vsq_call = pl.pallas_call(
    _vsq_body,
    grid=(GE,),
    in_specs=[pl.BlockSpec((EB, DE), lambda i: (i, 0)),
              pl.BlockSpec((D, DE), lambda i: (0, 0))],
    out_specs=_full,
    out_shape=jax.ShapeDtypeStruct((1, D), jnp.float32),
)

_stats_call = pl.pallas_call(
    _stats_body,
    grid=(GN,),
    in_specs=[_blkN, _blkN, _blk3, _blk3_16, _blk3_16, _blk3_16],
    out_specs=[_full,
               pl.BlockSpec((D, DE), lambda i: (0, 0)),
               _full,
               pl.BlockSpec((D, D), lambda i: (0, 0)),
               _full16],
    out_shape=[jax.ShapeDtypeStruct((1, D), jnp.float32),
               jax.ShapeDtypeStruct((D, DE), jnp.float32),
               jax.ShapeDtypeStruct((1, D), jnp.float32),
               jax.ShapeDtypeStruct((D, D), jnp.float32),
               jax.ShapeDtypeStruct((1, DE), jnp.float32)],
)

_final_call = pl.pallas_call(
    _final_body,
    grid=(GN,),
    in_specs=[_blk3, _blk3_16, _blk3_16, _blkN,
              _full, _full,
              pl.BlockSpec((D, DE), lambda i: (0, 0)),
              _full,
              pl.BlockSpec((D, D), lambda i: (0, 0)),
              _full16, _full,
              pl.BlockSpec((D, DE), lambda i: (0, 0)),
              pl.BlockSpec((D, D), lambda i: (0, 0)),
              _full, _full, _full, _full, _full],
    out_specs=_blkN,
    out_shape=jax.ShapeDtypeStruct((N, D), jnp.float32),
)


@jax.jit
def _run(atom_fea, nbr_dist_fea, self_fea_idx, nbr_fea_idx,
         W1, b1, g1, be1, W2, b2, g2, be2):
    w1a = W1[:, :D]
    w1b = W1[:, D:]
    u = _u_call(atom_fea, w1a, b1.reshape(1, D))
    vsq = _vsq_call(nbr_dist_fea, w1b)
    self_r = self_fea_idx.reshape(NW, NCHUNK, KC)
    nbr_r = nbr_fea_idx.reshape(NW, NCHUNK, KC)
    dist_r = nbr_dist_fea.reshape(NW, NCHUNK, KC, DE)
    gu2, sd2, z2, cnt2, sq2 = _sc_segment(u, dist_r, self_r, nbr_r)
    susq = sq2.reshape(NC, NS, 8, DE).sum(axis=(0, 1)).reshape(1, D)
    su, m2m, acol, s2g, sdcol = _stats_call(u, atom_fea, gu2, z2, cnt2, sd2)
    return _final_call(gu2, sd2, cnt2, atom_fea, su, susq, m2m, acol, s2g,
                       sdcol, vsq, w1b, W2, b2.reshape(1, D),
                       g1.reshape(1, D), be1.reshape(1, D),
                       g2.reshape(1, D), be2.reshape(1, D))


def kernel(atom_fea, nbr_dist_fea, nbr_adj_value, nbr_bond_type,
           self_fea_idx, nbr_fea_idx, ads_atom_idx,
           W1, b1, g1, be1, W2, b2, g2, be2):
    del nbr_adj_value, nbr_bond_type, ads_atom_idx
    return _run(atom_fea, nbr_dist_fea, self_fea_idx, nbr_fea_idx,
                W1, b1, g1, be1, W2, b2, g2, be2)
